# Initial kernel scaffold; baseline (speedup 1.0000x reference)
import functools

import jax
import jax.numpy as jnp
from jax import lax
from jax.experimental import pallas as pl
from jax.experimental.pallas import tpu as pltpu
from jax.experimental.pallas import tpu_sc as plsc

N = 10000
E = 320000
D = 128

NC, NS = 2, 16
NW = NC * NS
CHUNK = 128
CPW = 79
E_PAD = NW * CPW * CHUNK
N_ACC = 10240
RPT = N_ACC // NS

_mesh = plsc.VectorSubcoreMesh(core_axis_name="c", subcore_axis_name="s")


@functools.partial(
    pl.kernel,
    out_type=jax.ShapeDtypeStruct((NC, N_ACC, 16), jnp.float32),
    mesh=_mesh,
    scratch_types=[
        pltpu.VMEM((CPW, CHUNK), jnp.int32),
        pltpu.VMEM((CHUNK, 16), jnp.float32),
        pltpu.VMEM((CHUNK, 16), jnp.float32),
        pltpu.VMEM_SHARED((N_ACC, 16), jnp.float32),
    ],
)
def _sc_bincount(row_hbm, deg_out, row_idx_v, ones_v, zeros_v, deg_acc):
    cid = lax.axis_index("c")
    sid = lax.axis_index("s")
    wid = sid * NC + cid

    def _fill(i, _):
        ones_v[i] = jnp.ones((16,), jnp.float32)
        zeros_v[i] = jnp.zeros((16,), jnp.float32)
        return 0
    lax.fori_loop(0, CHUNK, _fill, 0)

    def _zero(j, _):
        pltpu.sync_copy(zeros_v, deg_acc.at[pl.ds(sid * RPT + j * CHUNK, CHUNK)])
        return 0
    lax.fori_loop(0, RPT // CHUNK, _zero, 0)
    plsc.subcore_barrier()

    pltpu.sync_copy(row_hbm.at[wid], row_idx_v)

    def _accum(j, _):
        pltpu.sync_copy(ones_v, deg_acc.at[row_idx_v.at[j]], add=True)
        return 0
    lax.fori_loop(0, CPW, _accum, 0)
    plsc.subcore_barrier()

    pltpu.sync_copy(deg_acc.at[pl.ds(sid * RPT, RPT)],
                    deg_out.at[cid, pl.ds(sid * RPT, RPT)])


def _scale_mm_body(dp_ref, x_ref, w_ref, o_ref):
    deg = dp_ref[0] + dp_ref[1]
    dis = lax.rsqrt(deg[:, :1])
    o_ref[...] = dis * jnp.dot(x_ref[...], w_ref[...],
                               preferred_element_type=jnp.float32)


def _scale_mm(dp, x_pad, weight):
    blk = 256
    grid = (N_ACC // blk,)
    return pl.pallas_call(
        _scale_mm_body,
        grid=grid,
        in_specs=[
            pl.BlockSpec((NC, blk, 16), lambda i: (0, i, 0)),
            pl.BlockSpec((blk, D), lambda i: (i, 0)),
            pl.BlockSpec((D, D), lambda i: (0, 0)),
        ],
        out_specs=pl.BlockSpec((blk, D), lambda i: (i, 0)),
        out_shape=jax.ShapeDtypeStruct((N_ACC, D), jnp.float32),
    )(dp, x_pad, weight)


@functools.partial(
    pl.kernel,
    out_type=jax.ShapeDtypeStruct((NC, N_ACC, D), jnp.float32),
    mesh=_mesh,
    scratch_types=[
        pltpu.VMEM((CPW, CHUNK), jnp.int32),
        pltpu.VMEM((CPW, CHUNK), jnp.int32),
        pltpu.VMEM((CHUNK, D), jnp.float32),
        pltpu.VMEM((CHUNK, D), jnp.float32),
        pltpu.SemaphoreType.DMA,
        pltpu.VMEM_SHARED((N_ACC, D), jnp.float32),
    ],
)
def _sc_scatter(scaled_hbm, row_hbm, col_hbm, out_hbm,
                row_idx_v, col_idx_v, rows_v, zeros_v, sem, acc):
    cid = lax.axis_index("c")
    sid = lax.axis_index("s")
    wid = sid * NC + cid

    def _fill(k, _):
        zeros_v[k // 8, pl.ds((k % 8) * 16, 16)] = jnp.zeros((16,), jnp.float32)
        return 0
    lax.fori_loop(0, CHUNK * D // 16, _fill, 0)

    def _zero(j, _):
        pltpu.sync_copy(zeros_v, acc.at[pl.ds(sid * RPT + j * CHUNK, CHUNK)])
        return 0
    lax.fori_loop(0, RPT // CHUNK, _zero, 0)
    plsc.subcore_barrier()

    pltpu.sync_copy(row_hbm.at[wid], row_idx_v)
    pltpu.sync_copy(col_hbm.at[wid], col_idx_v)

    def _edge_chunk(j, _):
        pltpu.async_copy(scaled_hbm.at[col_idx_v.at[j]], rows_v, sem).wait()
        pltpu.sync_copy(rows_v, acc.at[row_idx_v.at[j]], add=True)
        return 0
    lax.fori_loop(0, CPW, _edge_chunk, 0)
    plsc.subcore_barrier()

    pltpu.sync_copy(acc.at[pl.ds(sid * RPT, RPT)],
                    out_hbm.at[cid, pl.ds(sid * RPT, RPT)])


def _final_body(dp_ref, ap_ref, b_ref, o_ref):
    deg = (dp_ref[0] + dp_ref[1])[:, :1]
    dis = jnp.where(deg > 0, lax.rsqrt(deg), 0.0)
    o_ref[...] = dis * (ap_ref[0] + ap_ref[1]) + b_ref[...]


def _finalize(dp, ap, bias2d):
    blk = 1000
    grid = (N // blk,)
    return pl.pallas_call(
        _final_body,
        grid=grid,
        in_specs=[
            pl.BlockSpec((NC, blk, 16), lambda i: (0, i, 0)),
            pl.BlockSpec((NC, blk, D), lambda i: (0, i, 0)),
            pl.BlockSpec((1, D), lambda i: (0, 0)),
        ],
        out_specs=pl.BlockSpec((blk, D), lambda i: (i, 0)),
        out_shape=jax.ShapeDtypeStruct((N, D), jnp.float32),
    )(dp, ap, bias2d)


def kernel(x, edge_index, weight, bias):
    row = edge_index[0].astype(jnp.int32)
    col = edge_index[1].astype(jnp.int32)
    pad = jnp.full((E_PAD - E,), N, dtype=jnp.int32)
    row_p = jnp.concatenate([row, pad]).reshape(NW, CPW, CHUNK)
    col_p = jnp.concatenate([col, pad]).reshape(NW, CPW, CHUNK)
    x_pad = jnp.concatenate(
        [x, jnp.zeros((N_ACC - N, D), jnp.float32)], axis=0)

    dp = _sc_bincount(row_p)
    scaled = _scale_mm(dp, x_pad, weight)
    ap = _sc_scatter(scaled, row_p, col_p)
    return _finalize(dp, ap, bias.reshape(1, D))


# speedup vs baseline: 15.3667x; 15.3667x over previous
import functools

import jax
import jax.numpy as jnp
from jax import lax
from jax.experimental import pallas as pl
from jax.experimental.pallas import tpu as pltpu
from jax.experimental.pallas import tpu_sc as plsc

N = 10000
E = 320000
D = 128

NC, NS = 2, 16
NW = NC * NS
CHUNK = 128
CPW = 79
E_PAD = NW * CPW * CHUNK
N_ACC = 10240
RPT = N_ACC // NS

_mesh = plsc.VectorSubcoreMesh(core_axis_name="c", subcore_axis_name="s")


@functools.partial(
    pl.kernel,
    out_type=jax.ShapeDtypeStruct((NC, N_ACC, 16), jnp.float32),
    mesh=_mesh,
    scratch_types=[
        pltpu.VMEM((CPW, CHUNK), jnp.int32),
        pltpu.VMEM((CHUNK, 16), jnp.float32),
        pltpu.VMEM((CHUNK, 16), jnp.float32),
        pltpu.VMEM_SHARED((N_ACC, 16), jnp.float32),
    ],
)
def _sc_bincount(row_hbm, deg_out, row_idx_v, ones_v, zeros_v, deg_acc):
    cid = lax.axis_index("c")
    sid = lax.axis_index("s")
    wid = sid * NC + cid

    def _fill(i, _):
        ones_v[i] = jnp.ones((16,), jnp.float32)
        zeros_v[i] = jnp.zeros((16,), jnp.float32)
        return 0
    lax.fori_loop(0, CHUNK, _fill, 0)

    def _zero(j, _):
        pltpu.sync_copy(zeros_v, deg_acc.at[pl.ds(sid * RPT + j * CHUNK, CHUNK)])
        return 0
    lax.fori_loop(0, RPT // CHUNK, _zero, 0)
    plsc.subcore_barrier()

    pltpu.sync_copy(row_hbm.at[wid], row_idx_v)

    def _accum(j, _):
        pltpu.sync_copy(ones_v, deg_acc.at[row_idx_v.at[j]], add=True)
        return 0
    lax.fori_loop(0, CPW, _accum, 0)
    plsc.subcore_barrier()

    pltpu.sync_copy(deg_acc.at[pl.ds(sid * RPT, RPT)],
                    deg_out.at[cid, pl.ds(sid * RPT, RPT)])


def _scale_mm_body(dp_ref, x_ref, w_ref, o_ref):
    deg = dp_ref[0] + dp_ref[1]
    dis = lax.rsqrt(deg[:, :1])
    o_ref[...] = dis * jnp.dot(x_ref[...], w_ref[...],
                               preferred_element_type=jnp.float32)


def _scale_mm(dp, x_pad, weight):
    blk = 256
    grid = (N_ACC // blk,)
    return pl.pallas_call(
        _scale_mm_body,
        grid=grid,
        in_specs=[
            pl.BlockSpec((NC, blk, 16), lambda i: (0, i, 0)),
            pl.BlockSpec((blk, D), lambda i: (i, 0)),
            pl.BlockSpec((D, D), lambda i: (0, 0)),
        ],
        out_specs=pl.BlockSpec((blk, D), lambda i: (i, 0)),
        out_shape=jax.ShapeDtypeStruct((N_ACC, D), jnp.float32),
    )(dp, x_pad, weight)


@functools.partial(
    pl.kernel,
    out_type=jax.ShapeDtypeStruct((NC, N_ACC, D), jnp.float32),
    mesh=_mesh,
    scratch_types=[
        pltpu.VMEM((CPW, CHUNK), jnp.int32),
        pltpu.VMEM((CPW, CHUNK), jnp.int32),
        pltpu.VMEM((CHUNK, D), jnp.float32),
        pltpu.SemaphoreType.DMA,
        pltpu.VMEM_SHARED((N_ACC, D), jnp.float32),
    ],
)
def _sc_scatter(scaled_hbm, row_hbm, col_hbm, out_hbm,
                row_idx_v, col_idx_v, rows_v, sem, acc):
    cid = lax.axis_index("c")
    sid = lax.axis_index("s")
    wid = sid * NC + cid

    def _fill(k, _):
        rows_v[k // 8, pl.ds((k % 8) * 16, 16)] = jnp.zeros((16,), jnp.float32)
        return 0
    lax.fori_loop(0, CHUNK * D // 16, _fill, 0)

    def _zero(j, _):
        pltpu.sync_copy(rows_v, acc.at[pl.ds(sid * RPT + j * CHUNK, CHUNK)])
        return 0
    lax.fori_loop(0, RPT // CHUNK, _zero, 0)
    plsc.subcore_barrier()

    pltpu.sync_copy(row_hbm.at[wid], row_idx_v)
    pltpu.sync_copy(col_hbm.at[wid], col_idx_v)

    def _edge_chunk(j, _):
        pltpu.async_copy(scaled_hbm.at[col_idx_v.at[j]], rows_v, sem).wait()
        pltpu.sync_copy(rows_v, acc.at[row_idx_v.at[j]], add=True)
        return 0
    lax.fori_loop(0, CPW, _edge_chunk, 0)
    plsc.subcore_barrier()

    pltpu.sync_copy(acc.at[pl.ds(sid * RPT, RPT)],
                    out_hbm.at[cid, pl.ds(sid * RPT, RPT)])


def _final_body(dp_ref, ap_ref, b_ref, o_ref):
    deg = (dp_ref[0] + dp_ref[1])[:, :1]
    dis = jnp.where(deg > 0, lax.rsqrt(deg), 0.0)
    o_ref[...] = dis * (ap_ref[0] + ap_ref[1]) + b_ref[...]


def _finalize(dp, ap, bias2d):
    blk = 1000
    grid = (N // blk,)
    return pl.pallas_call(
        _final_body,
        grid=grid,
        in_specs=[
            pl.BlockSpec((NC, blk, 16), lambda i: (0, i, 0)),
            pl.BlockSpec((NC, blk, D), lambda i: (0, i, 0)),
            pl.BlockSpec((1, D), lambda i: (0, 0)),
        ],
        out_specs=pl.BlockSpec((blk, D), lambda i: (i, 0)),
        out_shape=jax.ShapeDtypeStruct((N, D), jnp.float32),
    )(dp, ap, bias2d)


def kernel(x, edge_index, weight, bias):
    row = edge_index[0].astype(jnp.int32)
    col = edge_index[1].astype(jnp.int32)
    pad = jnp.full((E_PAD - E,), N, dtype=jnp.int32)
    row_p = jnp.concatenate([row, pad]).reshape(NW, CPW, CHUNK)
    col_p = jnp.concatenate([col, pad]).reshape(NW, CPW, CHUNK)
    x_pad = jnp.concatenate(
        [x, jnp.zeros((N_ACC - N, D), jnp.float32)], axis=0)

    dp = _sc_bincount(row_p)
    scaled = _scale_mm(dp, x_pad, weight)
    ap = _sc_scatter(scaled, row_p, col_p)
    return _finalize(dp, ap, bias.reshape(1, D))
